# Initial kernel scaffold; baseline (speedup 1.0000x reference)
import jax
import jax.numpy as jnp
from jax.experimental import pallas as pl


def kernel(x, edge_index, edge_attr, num_graphs, W1, b1, W2, b2, root, root_b, gcn_w, gcn_b, f1w, f1b, f2w, f2b, f3w, f3b, pw, pb):
    raise NotImplementedError("write your pallas kernel here")


# speedup vs baseline: 1.6748x; 1.6748x over previous
import functools

import jax
import jax.numpy as jnp
from jax import lax
from jax.experimental import pallas as pl
from jax.experimental.pallas import tpu as pltpu
from jax.experimental.pallas import tpu_sc as plsc

N = 8192 * 6
E = 8192 * 5
NC, NS = 2, 16
NW = NC * NS
EPW = E // NW
EPT = E // NS
NPT = N // NS
FC = 32
IB = 128

_SC_PARAMS = pltpu.CompilerParams(use_tc_tiling_on_sc=False)


@functools.cache
def _sc_kernels():
    mesh = plsc.VectorSubcoreMesh(
        core_axis_name="c", subcore_axis_name="s",
        num_cores=NC, num_subcores=NS)
    g0 = _make_sc_gather_deg(mesh)
    s2 = _make_sc_scatter_msg(mesh)
    s3 = _make_sc_gcn(mesh)
    return g0, s2, s3


def _make_sc_gather_deg(mesh):
    return functools.partial(
        pl.kernel,
        out_type=(jax.ShapeDtypeStruct((E, 16), jnp.float32),
                  jax.ShapeDtypeStruct((N, 8), jnp.float32)),
        mesh=mesh,
        scratch_types=(
            pltpu.VMEM((EPW // IB, IB), jnp.int32),
            pltpu.VMEM((EPT // IB, IB), jnp.int32),
            pltpu.VMEM((IB, 16), jnp.float32),
            pltpu.VMEM((IB, 8), jnp.float32),
            pltpu.VMEM_SHARED((N, 8), jnp.float32),
            pltpu.SemaphoreType.DMA,
        ),
        compiler_params=_SC_PARAMS,
    )(_sc_gather_deg_body)


def _sc_gather_deg_body(xpad_hbm, src2d_hbm, dst2d_hbm, ones_hbm, zeros8_hbm,
                        xsrc_hbm, deg8_hbm,
                        sidx_v, didx_v, rows_v, ones_v, deg_sp, sem):
    cid = lax.axis_index("c")
    sid = lax.axis_index("s")
    wid = sid * NC + cid
    pltpu.sync_copy(src2d_hbm.at[wid], sidx_v)
    for j in range(EPW // IB):
        pltpu.async_copy(xpad_hbm.at[sidx_v.at[j]], rows_v, sem).wait()
        pltpu.sync_copy(rows_v, xsrc_hbm.at[pl.ds(wid * EPW + j * IB, IB)])

    @pl.when(cid == 0)
    def _deg():
        pltpu.sync_copy(zeros8_hbm, deg_sp.at[pl.ds(sid * NPT, NPT)])
        pltpu.sync_copy(dst2d_hbm.at[sid], didx_v)
        pltpu.sync_copy(ones_hbm, ones_v)
        plsc.subcore_barrier()
        for j in range(EPT // IB):
            pltpu.sync_copy(ones_v, deg_sp.at[didx_v.at[j]], add=True)
        plsc.subcore_barrier()
        pltpu.sync_copy(deg_sp.at[pl.ds(sid * NPT, NPT)],
                        deg8_hbm.at[pl.ds(sid * NPT, NPT)])


def _make_sc_scatter_msg(mesh):
    return functools.partial(
        pl.kernel,
        out_type=tuple(jax.ShapeDtypeStruct((N, FC), jnp.float32)
                       for _ in range(4)),
        mesh=mesh,
        scratch_types=(
            pltpu.VMEM((EPT // IB, IB), jnp.int32),
            pltpu.VMEM((IB, FC), jnp.float32),
            pltpu.VMEM_SHARED((N, FC), jnp.float32),
        ),
        compiler_params=_SC_PARAMS,
    )(_sc_scatter_msg_body)


def _sc_scatter_msg_body(dst2d_hbm, m0, m1, m2, m3, zeros32_hbm,
                         a0, a1, a2, a3,
                         didx_v, dat_v, acc_sp):
    cid = lax.axis_index("c")
    sid = lax.axis_index("s")
    msgs = (m0, m1, m2, m3)
    aggs = (a0, a1, a2, a3)
    pltpu.sync_copy(dst2d_hbm.at[sid], didx_v)
    for c in range(4):
        @pl.when(cid == c // 2)
        def _chunk(c=c):
            pltpu.sync_copy(zeros32_hbm, acc_sp.at[pl.ds(sid * NPT, NPT)])
            plsc.subcore_barrier()
            for j in range(EPT // IB):
                pltpu.sync_copy(
                    msgs[c].at[pl.ds(sid * EPT + j * IB, IB)], dat_v)
                pltpu.sync_copy(dat_v, acc_sp.at[didx_v.at[j]], add=True)
            plsc.subcore_barrier()
            pltpu.sync_copy(acc_sp.at[pl.ds(sid * NPT, NPT)],
                            aggs[c].at[pl.ds(sid * NPT, NPT)])


def _make_sc_gcn(mesh):
    return functools.partial(
        pl.kernel,
        out_type=tuple(jax.ShapeDtypeStruct((N, FC), jnp.float32)
                       for _ in range(4)),
        mesh=mesh,
        scratch_types=(
            pltpu.VMEM((EPT // IB, IB), jnp.int32),
            pltpu.VMEM((EPT // IB, IB), jnp.int32),
            pltpu.VMEM((IB, FC), jnp.float32),
            pltpu.VMEM_SHARED((N, FC), jnp.float32),
            pltpu.SemaphoreType.DMA,
        ),
        compiler_params=_SC_PARAMS,
    )(_sc_gcn_body)


def _sc_gcn_body(src2d_hbm, dst2d_hbm, n0, n1, n2, n3, zeros32_hbm,
                 s0, s1, s2, s3,
                 sidx_v, didx_v, dat_v, acc_sp, sem):
    cid = lax.axis_index("c")
    sid = lax.axis_index("s")
    nss = (n0, n1, n2, n3)
    outs = (s0, s1, s2, s3)
    pltpu.sync_copy(src2d_hbm.at[sid], sidx_v)
    pltpu.sync_copy(dst2d_hbm.at[sid], didx_v)
    for c in range(4):
        @pl.when(cid == c // 2)
        def _chunk(c=c):
            pltpu.sync_copy(zeros32_hbm, acc_sp.at[pl.ds(sid * NPT, NPT)])
            plsc.subcore_barrier()
            for j in range(EPT // IB):
                pltpu.async_copy(nss[c].at[sidx_v.at[j]], dat_v, sem).wait()
                pltpu.sync_copy(dat_v, acc_sp.at[didx_v.at[j]], add=True)
            plsc.subcore_barrier()
            pltpu.sync_copy(acc_sp.at[pl.ds(sid * NPT, NPT)],
                            outs[c].at[pl.ds(sid * NPT, NPT)])


EB = 2048


def _t1_body(ea_ref, xs_ref, w1_ref, b1_ref, w2_ref, b2_ref,
             o0, o1, o2, o3):
    h1 = jnp.maximum(
        jnp.dot(ea_ref[...], w1_ref[...],
                preferred_element_type=jnp.float32) + b1_ref[...], 0.0)
    xs = xs_ref[...]
    msg = jnp.zeros((EB, 128), jnp.float32)
    for i in range(7):
        h2i = jnp.maximum(
            jnp.dot(h1, w2_ref[:, 128 * i:128 * (i + 1)],
                    preferred_element_type=jnp.float32)
            + b2_ref[:, 128 * i:128 * (i + 1)], 0.0)
        msg = msg + xs[:, i:i + 1] * h2i
    o0[...] = msg[:, 0:32]
    o1[...] = msg[:, 32:64]
    o2[...] = msg[:, 64:96]
    o3[...] = msg[:, 96:128]


def _t1(ea, xsrc, W1, b1, W2, b2):
    f32 = jnp.float32
    return pl.pallas_call(
        _t1_body,
        grid=(E // EB,),
        in_specs=[
            pl.BlockSpec((EB, 19), lambda e: (e, 0)),
            pl.BlockSpec((EB, 16), lambda e: (e, 0)),
            pl.BlockSpec((19, 128), lambda e: (0, 0)),
            pl.BlockSpec((1, 128), lambda e: (0, 0)),
            pl.BlockSpec((128, 896), lambda e: (0, 0)),
            pl.BlockSpec((1, 896), lambda e: (0, 0)),
        ],
        out_specs=[pl.BlockSpec((EB, FC), lambda e: (e, 0))] * 4,
        out_shape=[jax.ShapeDtypeStruct((E, FC), f32)] * 4,
    )(ea, xsrc, W1, b1, W2, b2)


NB = 4096


def _t2_body(xp_ref, a0, a1, a2, a3, d8_ref, rw_ref, rb_ref,
             n0, n1, n2, n3, selfn_ref):
    node = (jnp.dot(xp_ref[...], rw_ref[...],
                    preferred_element_type=jnp.float32)
            + rb_ref[...]
            + jnp.concatenate([a0[...], a1[...], a2[...], a3[...]], axis=1))
    dinv = lax.rsqrt(d8_ref[:, 0:1] + 1.0)
    ns = dinv * node
    selfn_ref[...] = dinv * ns
    n0[...] = ns[:, 0:32]
    n1[...] = ns[:, 32:64]
    n2[...] = ns[:, 64:96]
    n3[...] = ns[:, 96:128]


def _t2(xpad, aggs, deg8, rootpad, root_b):
    f32 = jnp.float32
    return pl.pallas_call(
        _t2_body,
        grid=(N // NB,),
        in_specs=[
            pl.BlockSpec((NB, 16), lambda n: (n, 0)),
            pl.BlockSpec((NB, FC), lambda n: (n, 0)),
            pl.BlockSpec((NB, FC), lambda n: (n, 0)),
            pl.BlockSpec((NB, FC), lambda n: (n, 0)),
            pl.BlockSpec((NB, FC), lambda n: (n, 0)),
            pl.BlockSpec((NB, 8), lambda n: (n, 0)),
            pl.BlockSpec((16, 128), lambda n: (0, 0)),
            pl.BlockSpec((1, 128), lambda n: (0, 0)),
        ],
        out_specs=[pl.BlockSpec((NB, FC), lambda n: (n, 0))] * 4
        + [pl.BlockSpec((NB, 128), lambda n: (n, 0))],
        out_shape=[jax.ShapeDtypeStruct((N, FC), f32)] * 4
        + [jax.ShapeDtypeStruct((N, 128), f32)],
    )(xpad, *aggs, deg8, rootpad, root_b)


def _t3_body(s0, s1, s2, s3, selfn_ref, d8_ref, gw_ref, gb_ref, out_ref):
    S = jnp.concatenate([s0[...], s1[...], s2[...], s3[...]], axis=1)
    dinv = lax.rsqrt(d8_ref[:, 0:1] + 1.0)
    pre = dinv * S + selfn_ref[...]
    out_ref[...] = jnp.maximum(
        jnp.dot(pre, gw_ref[...], preferred_element_type=jnp.float32)
        + gb_ref[...], 0.0)


def _t3(Ss, selfn, deg8, gcn_w, gcn_b):
    return pl.pallas_call(
        _t3_body,
        grid=(N // NB,),
        in_specs=[
            pl.BlockSpec((NB, FC), lambda n: (n, 0)),
            pl.BlockSpec((NB, FC), lambda n: (n, 0)),
            pl.BlockSpec((NB, FC), lambda n: (n, 0)),
            pl.BlockSpec((NB, FC), lambda n: (n, 0)),
            pl.BlockSpec((NB, 128), lambda n: (n, 0)),
            pl.BlockSpec((NB, 8), lambda n: (n, 0)),
            pl.BlockSpec((128, 256), lambda n: (0, 0)),
            pl.BlockSpec((1, 256), lambda n: (0, 0)),
        ],
        out_specs=pl.BlockSpec((NB, 256), lambda n: (n, 0)),
        out_shape=jax.ShapeDtypeStruct((N, 256), jnp.float32),
    )(*Ss, selfn, deg8, gcn_w, gcn_b)


GB = 1024


def _t4_body(g_ref, w1, b1, w2, b2, w3, b3, pw, pb, out_ref):
    t = jnp.maximum(
        jnp.dot(g_ref[...], w1[...], preferred_element_type=jnp.float32)
        + b1[...], 0.0)
    t = jnp.maximum(
        jnp.dot(t, w2[...], preferred_element_type=jnp.float32) + b2[...],
        0.0)
    t = jnp.maximum(
        jnp.dot(t, w3[...], preferred_element_type=jnp.float32) + b3[...],
        0.0)
    out_ref[...] = (jnp.dot(t, pw[...], preferred_element_type=jnp.float32)
                    + pb[...])


def _t4(g, f1w, f1b, f2w, f2b, f3w, f3b, pw, pb):
    NG = N // 6
    return pl.pallas_call(
        _t4_body,
        grid=(NG // GB,),
        in_specs=[
            pl.BlockSpec((GB, 1536), lambda n: (n, 0)),
            pl.BlockSpec((1536, 512), lambda n: (0, 0)),
            pl.BlockSpec((1, 512), lambda n: (0, 0)),
            pl.BlockSpec((512, 256), lambda n: (0, 0)),
            pl.BlockSpec((1, 256), lambda n: (0, 0)),
            pl.BlockSpec((256, 128), lambda n: (0, 0)),
            pl.BlockSpec((1, 128), lambda n: (0, 0)),
            pl.BlockSpec((128, 7), lambda n: (0, 0)),
            pl.BlockSpec((1, 7), lambda n: (0, 0)),
        ],
        out_specs=pl.BlockSpec((GB, 7), lambda n: (n, 0)),
        out_shape=jax.ShapeDtypeStruct((NG, 7), jnp.float32),
    )(g, f1w, f1b, f2w, f2b, f3w, f3b, pw, pb)


def kernel(x, edge_index, edge_attr, num_graphs, W1, b1, W2, b2, root,
           root_b, gcn_w, gcn_b, f1w, f1b, f2w, f2b, f3w, f3b, pw, pb):
    f32 = jnp.float32
    src = edge_index[0].astype(jnp.int32)
    dst = edge_index[1].astype(jnp.int32)
    src3d_w = src.reshape(NW, EPW // IB, IB)
    src3d = src.reshape(NS, EPT // IB, IB)
    dst3d = dst.reshape(NS, EPT // IB, IB)
    xpad = jnp.pad(x, ((0, 0), (0, 9)))
    rootpad = jnp.pad(root, ((0, 9), (0, 0)))
    ones8 = jnp.ones((IB, 8), f32)
    zeros8 = jnp.zeros((NPT, 8), f32)
    zeros32 = jnp.zeros((NPT, FC), f32)

    sc_gather_deg, sc_scatter_msg, sc_gcn = _sc_kernels()
    xsrc, deg8 = sc_gather_deg(xpad, src3d_w, dst3d, ones8, zeros8)
    msgs = _t1(edge_attr, xsrc, W1, b1.reshape(1, 128), W2,
               b2.reshape(1, 896))
    aggs = sc_scatter_msg(dst3d, *msgs, zeros32)
    nss_selfn = _t2(xpad, aggs, deg8, rootpad, root_b.reshape(1, 128))
    nss, selfn = nss_selfn[:4], nss_selfn[4]
    Ss = sc_gcn(src3d, dst3d, *nss, zeros32)
    h2g = _t3(Ss, selfn, deg8, gcn_w, gcn_b.reshape(1, 256))
    g = h2g.reshape(N // 6, 1536)
    return _t4(g, f1w, f1b.reshape(1, 512), f2w, f2b.reshape(1, 256),
               f3w, f3b.reshape(1, 128), pw, pb.reshape(1, 7))


# speedup vs baseline: 2.4564x; 1.4666x over previous
import functools

import jax
import jax.numpy as jnp
from jax import lax
from jax.experimental import pallas as pl
from jax.experimental.pallas import tpu as pltpu
from jax.experimental.pallas import tpu_sc as plsc

N = 8192 * 6
E = 8192 * 5
NC, NS = 2, 16
NW = NC * NS
EPW = E // NW
EPT = E // NS
NPT = N // NS
IB = 128
NRANGE = 4
NR = N // NRANGE
RPT = NR // NS

_SC_PARAMS = pltpu.CompilerParams(use_tc_tiling_on_sc=False)


@functools.cache
def _sc_kernels():
    mesh = plsc.VectorSubcoreMesh(
        core_axis_name="c", subcore_axis_name="s",
        num_cores=NC, num_subcores=NS)
    g0 = _make_sc_gather_deg(mesh)
    s2 = _make_sc_scatter_msg(mesh)
    s3 = _make_sc_gcn(mesh)
    return g0, s2, s3


def _make_sc_gather_deg(mesh):
    return functools.partial(
        pl.kernel,
        out_type=(jax.ShapeDtypeStruct((E, 128), jnp.float32),
                  jax.ShapeDtypeStruct((N, 128), jnp.float32)),
        mesh=mesh,
        scratch_types=(
            pltpu.VMEM((EPW // IB, IB), jnp.int32),
            pltpu.VMEM((EPT // IB, IB), jnp.int32),
            pltpu.VMEM((IB, 128), jnp.float32),
            pltpu.VMEM((IB, 8), jnp.float32),
            pltpu.VMEM_SHARED((N, 8), jnp.float32),
            pltpu.SemaphoreType.DMA,
        ),
        compiler_params=_SC_PARAMS,
    )(_sc_gather_deg_body)


def _sc_gather_deg_body(xpad_hbm, src3d_hbm, dst3d_hbm, ones_hbm, zeros8_hbm,
                        xsrc_hbm, deg_hbm,
                        sidx_v, didx_v, rows_v, ones_v, deg_sp, sem):
    cid = lax.axis_index("c")
    sid = lax.axis_index("s")
    wid = sid * NC + cid
    pltpu.sync_copy(src3d_hbm.at[wid], sidx_v)
    for j in range(EPW // IB):
        pltpu.async_copy(xpad_hbm.at[sidx_v.at[j]], rows_v, sem).wait()
        pltpu.sync_copy(rows_v, xsrc_hbm.at[pl.ds(wid * EPW + j * IB, IB)])

    @pl.when(cid == 0)
    def _deg():
        pltpu.sync_copy(zeros8_hbm, deg_sp.at[pl.ds(sid * NPT, NPT)])
        pltpu.sync_copy(dst3d_hbm.at[sid], didx_v)
        pltpu.sync_copy(ones_hbm, ones_v)
        plsc.subcore_barrier()
        for j in range(EPT // IB):
            pltpu.sync_copy(ones_v, deg_sp.at[didx_v.at[j]], add=True)
        plsc.subcore_barrier()
        pltpu.sync_copy(deg_sp.at[pl.ds(sid * NPT, NPT)],
                        deg_hbm.at[pl.ds(sid * NPT, NPT), pl.ds(0, 8)])


def _make_sc_scatter_msg(mesh):
    return functools.partial(
        pl.kernel,
        out_type=jax.ShapeDtypeStruct((N, 128), jnp.float32),
        mesh=mesh,
        scratch_types=(
            pltpu.VMEM((EPT // IB, IB), jnp.int32),
            pltpu.VMEM((EPT // IB, IB), jnp.int32),
            pltpu.VMEM((IB, 128), jnp.float32),
            pltpu.VMEM_SHARED((NR + 8, 128), jnp.float32),
        ),
        compiler_params=_SC_PARAMS,
    )(_sc_scatter_msg_body)


def _rewrite_range_idx(didx_v, didx_r, lo):
    hi = lo + NR
    for r in range(EPT // IB):
        for k in range(IB // 16):
            v = didx_v[r, pl.ds(16 * k, 16)]
            ok = (v >= lo) & (v < hi)
            didx_r[r, pl.ds(16 * k, 16)] = jnp.where(
                ok, v - lo, jnp.full((16,), NR, jnp.int32))


def _sc_scatter_msg_body(dst3d_hbm, msg_hbm, zeros_hbm, agg_hbm,
                         didx_v, didx_r, dat_v, acc_sp):
    cid = lax.axis_index("c")
    sid = lax.axis_index("s")
    pltpu.sync_copy(dst3d_hbm.at[sid], didx_v)
    for p in range(2):
        rng = 2 * cid + p
        lo = rng * NR
        _rewrite_range_idx(didx_v, didx_r, lo)
        pltpu.sync_copy(zeros_hbm, acc_sp.at[pl.ds(sid * RPT, RPT)])
        plsc.subcore_barrier()
        for j in range(EPT // IB):
            pltpu.sync_copy(msg_hbm.at[pl.ds(sid * EPT + j * IB, IB)], dat_v)
            pltpu.sync_copy(dat_v, acc_sp.at[didx_r.at[j]], add=True)
        plsc.subcore_barrier()
        pltpu.sync_copy(acc_sp.at[pl.ds(sid * RPT, RPT)],
                        agg_hbm.at[pl.ds(lo + sid * RPT, RPT)])
        plsc.subcore_barrier()


def _make_sc_gcn(mesh):
    return functools.partial(
        pl.kernel,
        out_type=jax.ShapeDtypeStruct((N, 128), jnp.float32),
        mesh=mesh,
        scratch_types=(
            pltpu.VMEM((EPT // IB, IB), jnp.int32),
            pltpu.VMEM((EPT // IB, IB), jnp.int32),
            pltpu.VMEM((EPT // IB, IB), jnp.int32),
            pltpu.VMEM((IB, 128), jnp.float32),
            pltpu.VMEM_SHARED((NR + 8, 128), jnp.float32),
            pltpu.SemaphoreType.DMA,
        ),
        compiler_params=_SC_PARAMS,
    )(_sc_gcn_body)


def _sc_gcn_body(src3d_hbm, dst3d_hbm, ns_hbm, zeros_hbm, s_hbm,
                 sidx_v, didx_v, didx_r, dat_v, acc_sp, sem):
    cid = lax.axis_index("c")
    sid = lax.axis_index("s")
    pltpu.sync_copy(src3d_hbm.at[sid], sidx_v)
    pltpu.sync_copy(dst3d_hbm.at[sid], didx_v)
    for p in range(2):
        rng = 2 * cid + p
        lo = rng * NR
        _rewrite_range_idx(didx_v, didx_r, lo)
        pltpu.sync_copy(zeros_hbm, acc_sp.at[pl.ds(sid * RPT, RPT)])
        plsc.subcore_barrier()
        for j in range(EPT // IB):
            pltpu.async_copy(ns_hbm.at[sidx_v.at[j]], dat_v, sem).wait()
            pltpu.sync_copy(dat_v, acc_sp.at[didx_r.at[j]], add=True)
        plsc.subcore_barrier()
        pltpu.sync_copy(acc_sp.at[pl.ds(sid * RPT, RPT)],
                        s_hbm.at[pl.ds(lo + sid * RPT, RPT)])
        plsc.subcore_barrier()


EB = 2048


def _t1_body(ea_ref, xs_ref, w1_ref, b1_ref, w2_ref, b2_ref, o_ref):
    h1 = jnp.maximum(
        jnp.dot(ea_ref[...], w1_ref[...],
                preferred_element_type=jnp.float32) + b1_ref[...], 0.0)
    xs = xs_ref[...]
    msg = jnp.zeros((EB, 128), jnp.float32)
    for i in range(7):
        h2i = jnp.maximum(
            jnp.dot(h1, w2_ref[:, 128 * i:128 * (i + 1)],
                    preferred_element_type=jnp.float32)
            + b2_ref[:, 128 * i:128 * (i + 1)], 0.0)
        msg = msg + xs[:, i:i + 1] * h2i
    o_ref[...] = msg


def _t1(ea, xsrc, W1, b1, W2, b2):
    return pl.pallas_call(
        _t1_body,
        grid=(E // EB,),
        in_specs=[
            pl.BlockSpec((EB, 19), lambda e: (e, 0)),
            pl.BlockSpec((EB, 128), lambda e: (e, 0)),
            pl.BlockSpec((19, 128), lambda e: (0, 0)),
            pl.BlockSpec((1, 128), lambda e: (0, 0)),
            pl.BlockSpec((128, 896), lambda e: (0, 0)),
            pl.BlockSpec((1, 896), lambda e: (0, 0)),
        ],
        out_specs=pl.BlockSpec((EB, 128), lambda e: (e, 0)),
        out_shape=jax.ShapeDtypeStruct((E, 128), jnp.float32),
    )(ea, xsrc, W1, b1, W2, b2)


NB = 4096


def _t2_body(xp_ref, agg_ref, deg_ref, rw_ref, rb_ref, ns_ref, selfn_ref):
    node = (jnp.dot(xp_ref[...], rw_ref[...],
                    preferred_element_type=jnp.float32)
            + rb_ref[...] + agg_ref[...])
    dinv = lax.rsqrt(deg_ref[:, 0:1] + 1.0)
    ns = dinv * node
    ns_ref[...] = ns
    selfn_ref[...] = dinv * ns


def _t2(xpad, agg, deg, rootpad, root_b):
    f32 = jnp.float32
    return pl.pallas_call(
        _t2_body,
        grid=(N // NB,),
        in_specs=[
            pl.BlockSpec((NB, 128), lambda n: (n, 0)),
            pl.BlockSpec((NB, 128), lambda n: (n, 0)),
            pl.BlockSpec((NB, 128), lambda n: (n, 0)),
            pl.BlockSpec((128, 128), lambda n: (0, 0)),
            pl.BlockSpec((1, 128), lambda n: (0, 0)),
        ],
        out_specs=[pl.BlockSpec((NB, 128), lambda n: (n, 0))] * 2,
        out_shape=[jax.ShapeDtypeStruct((N, 128), f32)] * 2,
    )(xpad, agg, deg, rootpad, root_b)


def _t3_body(s_ref, selfn_ref, deg_ref, gw_ref, gb_ref, out_ref):
    dinv = lax.rsqrt(deg_ref[:, 0:1] + 1.0)
    pre = dinv * s_ref[...] + selfn_ref[...]
    out_ref[...] = jnp.maximum(
        jnp.dot(pre, gw_ref[...], preferred_element_type=jnp.float32)
        + gb_ref[...], 0.0)


def _t3(S, selfn, deg, gcn_w, gcn_b):
    return pl.pallas_call(
        _t3_body,
        grid=(N // NB,),
        in_specs=[
            pl.BlockSpec((NB, 128), lambda n: (n, 0)),
            pl.BlockSpec((NB, 128), lambda n: (n, 0)),
            pl.BlockSpec((NB, 128), lambda n: (n, 0)),
            pl.BlockSpec((128, 256), lambda n: (0, 0)),
            pl.BlockSpec((1, 256), lambda n: (0, 0)),
        ],
        out_specs=pl.BlockSpec((NB, 256), lambda n: (n, 0)),
        out_shape=jax.ShapeDtypeStruct((N, 256), jnp.float32),
    )(S, selfn, deg, gcn_w, gcn_b)


GB = 1024


def _t4_body(g_ref, w1, b1, w2, b2, w3, b3, pw, pb, out_ref):
    t = jnp.maximum(
        jnp.dot(g_ref[...], w1[...], preferred_element_type=jnp.float32)
        + b1[...], 0.0)
    t = jnp.maximum(
        jnp.dot(t, w2[...], preferred_element_type=jnp.float32) + b2[...],
        0.0)
    t = jnp.maximum(
        jnp.dot(t, w3[...], preferred_element_type=jnp.float32) + b3[...],
        0.0)
    out_ref[...] = (jnp.dot(t, pw[...], preferred_element_type=jnp.float32)
                    + pb[...])


def _t4(g, f1w, f1b, f2w, f2b, f3w, f3b, pw, pb):
    NG = N // 6
    return pl.pallas_call(
        _t4_body,
        grid=(NG // GB,),
        in_specs=[
            pl.BlockSpec((GB, 1536), lambda n: (n, 0)),
            pl.BlockSpec((1536, 512), lambda n: (0, 0)),
            pl.BlockSpec((1, 512), lambda n: (0, 0)),
            pl.BlockSpec((512, 256), lambda n: (0, 0)),
            pl.BlockSpec((1, 256), lambda n: (0, 0)),
            pl.BlockSpec((256, 128), lambda n: (0, 0)),
            pl.BlockSpec((1, 128), lambda n: (0, 0)),
            pl.BlockSpec((128, 7), lambda n: (0, 0)),
            pl.BlockSpec((1, 7), lambda n: (0, 0)),
        ],
        out_specs=pl.BlockSpec((GB, 7), lambda n: (n, 0)),
        out_shape=jax.ShapeDtypeStruct((NG, 7), jnp.float32),
    )(g, f1w, f1b, f2w, f2b, f3w, f3b, pw, pb)


def kernel(x, edge_index, edge_attr, num_graphs, W1, b1, W2, b2, root,
           root_b, gcn_w, gcn_b, f1w, f1b, f2w, f2b, f3w, f3b, pw, pb):
    f32 = jnp.float32
    src = edge_index[0].astype(jnp.int32)
    dst = edge_index[1].astype(jnp.int32)
    src3d_w = src.reshape(NW, EPW // IB, IB)
    src3d = src.reshape(NS, EPT // IB, IB)
    dst3d = dst.reshape(NS, EPT // IB, IB)
    xpad = jnp.pad(x, ((0, 0), (0, 121)))
    rootpad = jnp.pad(root, ((0, 121), (0, 0)))
    ones8 = jnp.ones((IB, 8), f32)
    zeros8 = jnp.zeros((NPT, 8), f32)
    zerosr = jnp.zeros((RPT, 128), f32)

    sc_gather_deg, sc_scatter_msg, sc_gcn = _sc_kernels()
    xsrc, deg = sc_gather_deg(xpad, src3d_w, dst3d, ones8, zeros8)
    msg = _t1(edge_attr, xsrc, W1, b1.reshape(1, 128), W2,
              b2.reshape(1, 896))
    agg = sc_scatter_msg(dst3d, msg, zerosr)
    ns, selfn = _t2(xpad, agg, deg, rootpad, root_b.reshape(1, 128))
    S = sc_gcn(src3d, dst3d, ns, zerosr)
    h2g = _t3(S, selfn, deg, gcn_w, gcn_b.reshape(1, 256))
    g = h2g.reshape(N // 6, 1536)
    return _t4(g, f1w, f1b.reshape(1, 512), f2w, f2b.reshape(1, 256),
               f3w, f3b.reshape(1, 128), pw, pb.reshape(1, 7))


# speedup vs baseline: 2.5065x; 1.0204x over previous
import functools

import jax
import jax.numpy as jnp
from jax import lax
from jax.experimental import pallas as pl
from jax.experimental.pallas import tpu as pltpu
from jax.experimental.pallas import tpu_sc as plsc

N = 8192 * 6
E = 8192 * 5
NC, NS = 2, 16
NW = NC * NS
EPW = E // NW
EPT = E // NS
NPT = N // NS
IB = 128
NRANGE = 4
NR = N // NRANGE
RPT = NR // NS

_SC_PARAMS = pltpu.CompilerParams(use_tc_tiling_on_sc=False)


@functools.cache
def _sc_kernels():
    mesh = plsc.VectorSubcoreMesh(
        core_axis_name="c", subcore_axis_name="s",
        num_cores=NC, num_subcores=NS)
    g0 = _make_sc_gather_deg(mesh)
    s2 = _make_sc_scatter_msg(mesh)
    s3 = _make_sc_gcn(mesh)
    return g0, s2, s3


def _make_sc_gather_deg(mesh):
    return functools.partial(
        pl.kernel,
        out_type=(jax.ShapeDtypeStruct((E, 128), jnp.float32),
                  jax.ShapeDtypeStruct((N, 128), jnp.float32)),
        mesh=mesh,
        scratch_types=(
            pltpu.VMEM((EPW // IB, IB), jnp.int32),
            pltpu.VMEM((EPT // IB, IB), jnp.int32),
            pltpu.VMEM((IB, 128), jnp.float32),
            pltpu.VMEM((IB, 8), jnp.float32),
            pltpu.VMEM_SHARED((N, 8), jnp.float32),
            pltpu.SemaphoreType.DMA,
        ),
        compiler_params=_SC_PARAMS,
    )(_sc_gather_deg_body)


def _sc_gather_deg_body(xpad_hbm, src3d_hbm, dst3d_hbm, ones_hbm, zeros8_hbm,
                        xsrc_hbm, deg_hbm,
                        sidx_v, didx_v, rows_v, ones_v, deg_sp, sem):
    cid = lax.axis_index("c")
    sid = lax.axis_index("s")
    wid = sid * NC + cid
    pltpu.sync_copy(src3d_hbm.at[wid], sidx_v)
    for j in range(EPW // IB):
        pltpu.async_copy(xpad_hbm.at[sidx_v.at[j]], rows_v, sem).wait()
        pltpu.sync_copy(rows_v, xsrc_hbm.at[pl.ds(wid * EPW + j * IB, IB)])

    @pl.when(cid == 0)
    def _deg():
        pltpu.sync_copy(zeros8_hbm, deg_sp.at[pl.ds(sid * NPT, NPT)])
        pltpu.sync_copy(dst3d_hbm.at[sid], didx_v)
        pltpu.sync_copy(ones_hbm, ones_v)
        plsc.subcore_barrier()
        for j in range(EPT // IB):
            pltpu.sync_copy(ones_v, deg_sp.at[didx_v.at[j]], add=True)
        plsc.subcore_barrier()
        pltpu.sync_copy(deg_sp.at[pl.ds(sid * NPT, NPT)],
                        deg_hbm.at[pl.ds(sid * NPT, NPT), pl.ds(0, 8)])


def _make_sc_scatter_msg(mesh):
    return functools.partial(
        pl.kernel,
        out_type=jax.ShapeDtypeStruct((N, 128), jnp.float32),
        mesh=mesh,
        scratch_types=(
            pltpu.VMEM((EPT // IB, IB), jnp.int32),
            pltpu.VMEM((EPT // IB, IB), jnp.int32),
            pltpu.VMEM((IB, 128), jnp.float32),
            pltpu.VMEM_SHARED((NR + 8, 128), jnp.float32),
        ),
        compiler_params=_SC_PARAMS,
    )(_sc_scatter_msg_body)


def _rewrite_range_idx(didx_v, didx_r, lo):
    hi = lo + NR
    for r in range(EPT // IB):
        for k in range(IB // 16):
            v = didx_v[r, pl.ds(16 * k, 16)]
            ok = (v >= lo) & (v < hi)
            didx_r[r, pl.ds(16 * k, 16)] = jnp.where(
                ok, v - lo, jnp.full((16,), NR, jnp.int32))


def _sc_scatter_msg_body(dst3d_hbm, msg_hbm, zeros_hbm, agg_hbm,
                         didx_v, didx_r, dat_v, acc_sp):
    cid = lax.axis_index("c")
    sid = lax.axis_index("s")
    pltpu.sync_copy(dst3d_hbm.at[sid], didx_v)
    for p in range(2):
        rng = 2 * cid + p
        lo = rng * NR
        _rewrite_range_idx(didx_v, didx_r, lo)
        pltpu.sync_copy(zeros_hbm, acc_sp.at[pl.ds(sid * RPT, RPT)])
        plsc.subcore_barrier()
        for j in range(EPT // IB):
            pltpu.sync_copy(msg_hbm.at[pl.ds(sid * EPT + j * IB, IB)], dat_v)
            pltpu.sync_copy(dat_v, acc_sp.at[didx_r.at[j]], add=True)
        plsc.subcore_barrier()
        pltpu.sync_copy(acc_sp.at[pl.ds(sid * RPT, RPT)],
                        agg_hbm.at[pl.ds(lo + sid * RPT, RPT)])
        plsc.subcore_barrier()


def _make_sc_gcn(mesh):
    return functools.partial(
        pl.kernel,
        out_type=jax.ShapeDtypeStruct((N, 128), jnp.float32),
        mesh=mesh,
        scratch_types=(
            pltpu.VMEM((EPT // IB, IB), jnp.int32),
            pltpu.VMEM((EPT // IB, IB), jnp.int32),
            pltpu.VMEM((EPT // IB, IB), jnp.int32),
            pltpu.VMEM((IB, 128), jnp.float32),
            pltpu.VMEM_SHARED((NR + 8, 128), jnp.float32),
            pltpu.SemaphoreType.DMA,
        ),
        compiler_params=_SC_PARAMS,
    )(_sc_gcn_body)


def _sc_gcn_body(src3d_hbm, dst3d_hbm, ns_hbm, zeros_hbm, s_hbm,
                 sidx_v, didx_v, didx_r, dat_v, acc_sp, sem):
    cid = lax.axis_index("c")
    sid = lax.axis_index("s")
    pltpu.sync_copy(src3d_hbm.at[sid], sidx_v)
    pltpu.sync_copy(dst3d_hbm.at[sid], didx_v)
    for p in range(2):
        rng = 2 * cid + p
        lo = rng * NR
        _rewrite_range_idx(didx_v, didx_r, lo)
        pltpu.sync_copy(zeros_hbm, acc_sp.at[pl.ds(sid * RPT, RPT)])
        plsc.subcore_barrier()
        for j in range(EPT // IB):
            pltpu.async_copy(ns_hbm.at[sidx_v.at[j]], dat_v, sem).wait()
            pltpu.sync_copy(dat_v, acc_sp.at[didx_r.at[j]], add=True)
        plsc.subcore_barrier()
        pltpu.sync_copy(acc_sp.at[pl.ds(sid * RPT, RPT)],
                        s_hbm.at[pl.ds(lo + sid * RPT, RPT)])
        plsc.subcore_barrier()


EB = 2048


def _t1_body(ea_ref, xs_ref, w1_ref, b1_ref, w2_ref, b2_ref, o_ref):
    h1 = jnp.maximum(
        jnp.dot(ea_ref[...], w1_ref[...],
                preferred_element_type=jnp.float32) + b1_ref[...], 0.0)
    xs = xs_ref[...]
    msg = jnp.zeros((EB, 128), jnp.float32)
    for i in range(7):
        h2i = jnp.maximum(
            jnp.dot(h1, w2_ref[:, 128 * i:128 * (i + 1)],
                    preferred_element_type=jnp.float32)
            + b2_ref[:, 128 * i:128 * (i + 1)], 0.0)
        msg = msg + xs[:, i:i + 1] * h2i
    o_ref[...] = msg


def _t1(ea, xsrc, W1, b1, W2, b2):
    return pl.pallas_call(
        _t1_body,
        grid=(E // EB,),
        in_specs=[
            pl.BlockSpec((EB, 19), lambda e: (e, 0)),
            pl.BlockSpec((EB, 128), lambda e: (e, 0)),
            pl.BlockSpec((19, 128), lambda e: (0, 0)),
            pl.BlockSpec((1, 128), lambda e: (0, 0)),
            pl.BlockSpec((128, 896), lambda e: (0, 0)),
            pl.BlockSpec((1, 896), lambda e: (0, 0)),
        ],
        out_specs=pl.BlockSpec((EB, 128), lambda e: (e, 0)),
        out_shape=jax.ShapeDtypeStruct((E, 128), jnp.float32),
    )(ea, xsrc, W1, b1, W2, b2)


NB = 4096


def _t2_body(xp_ref, agg_ref, deg_ref, rw_ref, rb_ref, ns_ref):
    node = (jnp.dot(xp_ref[...], rw_ref[...],
                    preferred_element_type=jnp.float32)
            + rb_ref[...] + agg_ref[...])
    dinv = lax.rsqrt(deg_ref[:, 0:1] + 1.0)
    ns_ref[...] = dinv * node


def _t2(xpad, agg, deg, rootpad, root_b):
    f32 = jnp.float32
    return pl.pallas_call(
        _t2_body,
        grid=(N // NB,),
        in_specs=[
            pl.BlockSpec((NB, 128), lambda n: (n, 0)),
            pl.BlockSpec((NB, 128), lambda n: (n, 0)),
            pl.BlockSpec((NB, 128), lambda n: (n, 0)),
            pl.BlockSpec((128, 128), lambda n: (0, 0)),
            pl.BlockSpec((1, 128), lambda n: (0, 0)),
        ],
        out_specs=pl.BlockSpec((NB, 128), lambda n: (n, 0)),
        out_shape=jax.ShapeDtypeStruct((N, 128), f32),
    )(xpad, agg, deg, rootpad, root_b)


def _t3_body(s_ref, ns_ref, deg_ref, gw_ref, gb_ref, out_ref):
    dinv = lax.rsqrt(deg_ref[:, 0:1] + 1.0)
    pre = dinv * (s_ref[...] + ns_ref[...])
    out_ref[...] = jnp.maximum(
        jnp.dot(pre, gw_ref[...], preferred_element_type=jnp.float32)
        + gb_ref[...], 0.0)


def _t3(S, ns, deg, gcn_w, gcn_b):
    return pl.pallas_call(
        _t3_body,
        grid=(N // NB,),
        in_specs=[
            pl.BlockSpec((NB, 128), lambda n: (n, 0)),
            pl.BlockSpec((NB, 128), lambda n: (n, 0)),
            pl.BlockSpec((NB, 128), lambda n: (n, 0)),
            pl.BlockSpec((128, 256), lambda n: (0, 0)),
            pl.BlockSpec((1, 256), lambda n: (0, 0)),
        ],
        out_specs=pl.BlockSpec((NB, 256), lambda n: (n, 0)),
        out_shape=jax.ShapeDtypeStruct((N, 256), jnp.float32),
    )(S, ns, deg, gcn_w, gcn_b)


GB = 1024


def _t4_body(g_ref, w1, b1, w2, b2, w3, b3, pw, pb, out_ref):
    t = jnp.maximum(
        jnp.dot(g_ref[...], w1[...], preferred_element_type=jnp.float32)
        + b1[...], 0.0)
    t = jnp.maximum(
        jnp.dot(t, w2[...], preferred_element_type=jnp.float32) + b2[...],
        0.0)
    t = jnp.maximum(
        jnp.dot(t, w3[...], preferred_element_type=jnp.float32) + b3[...],
        0.0)
    out_ref[...] = (jnp.dot(t, pw[...], preferred_element_type=jnp.float32)
                    + pb[...])


def _t4(g, f1w, f1b, f2w, f2b, f3w, f3b, pw, pb):
    NG = N // 6
    return pl.pallas_call(
        _t4_body,
        grid=(NG // GB,),
        in_specs=[
            pl.BlockSpec((GB, 1536), lambda n: (n, 0)),
            pl.BlockSpec((1536, 512), lambda n: (0, 0)),
            pl.BlockSpec((1, 512), lambda n: (0, 0)),
            pl.BlockSpec((512, 256), lambda n: (0, 0)),
            pl.BlockSpec((1, 256), lambda n: (0, 0)),
            pl.BlockSpec((256, 128), lambda n: (0, 0)),
            pl.BlockSpec((1, 128), lambda n: (0, 0)),
            pl.BlockSpec((128, 7), lambda n: (0, 0)),
            pl.BlockSpec((1, 7), lambda n: (0, 0)),
        ],
        out_specs=pl.BlockSpec((GB, 7), lambda n: (n, 0)),
        out_shape=jax.ShapeDtypeStruct((NG, 7), jnp.float32),
    )(g, f1w, f1b, f2w, f2b, f3w, f3b, pw, pb)


def kernel(x, edge_index, edge_attr, num_graphs, W1, b1, W2, b2, root,
           root_b, gcn_w, gcn_b, f1w, f1b, f2w, f2b, f3w, f3b, pw, pb):
    f32 = jnp.float32
    src = edge_index[0].astype(jnp.int32)
    dst = edge_index[1].astype(jnp.int32)
    src3d_w = src.reshape(NW, EPW // IB, IB)
    src3d = src.reshape(NS, EPT // IB, IB)
    dst3d = dst.reshape(NS, EPT // IB, IB)
    xpad = jnp.pad(x, ((0, 0), (0, 121)))
    rootpad = jnp.pad(root, ((0, 121), (0, 0)))
    ones8 = jnp.ones((IB, 8), f32)
    zeros8 = jnp.zeros((NPT, 8), f32)
    zerosr = jnp.zeros((RPT, 128), f32)

    sc_gather_deg, sc_scatter_msg, sc_gcn = _sc_kernels()
    xsrc, deg = sc_gather_deg(xpad, src3d_w, dst3d, ones8, zeros8)
    msg = _t1(edge_attr, xsrc, W1, b1.reshape(1, 128), W2,
              b2.reshape(1, 896))
    agg = sc_scatter_msg(dst3d, msg, zerosr)
    ns = _t2(xpad, agg, deg, rootpad, root_b.reshape(1, 128))
    S = sc_gcn(src3d, dst3d, ns, zerosr)
    h2g = _t3(S, ns, deg, gcn_w, gcn_b.reshape(1, 256))
    g = h2g.reshape(N // 6, 1536)
    return _t4(g, f1w, f1b.reshape(1, 512), f2w, f2b.reshape(1, 256),
               f3w, f3b.reshape(1, 128), pw, pb.reshape(1, 7))
